# baseline (speedup 1.0000x reference)
import functools
import math

import jax
import jax.numpy as jnp
from jax import lax
from jax.experimental import pallas as pl
from jax.experimental.pallas import tpu as pltpu
from jax.experimental.pallas import tpu_sc as plsc

_N = 10000
_G = 64
_E = 160000
_D = 128

_NC = 2
_NS = 16
_CH = 16
_NCHUNK = 8
_PASS = _NCHUNK // _NC
_EB = 80
_EPT = _E // _NS
_NBLK = _EPT // _EB
_NP = 10240
_ROWS = _NP // _NS
_PCOLS = 6 * _CH + 16

_BE = 10000
_BN = 1000
_S15 = math.sqrt(15.0)
_S5 = math.sqrt(5.0)
_INV_S4PI = 1.0 / math.sqrt(4.0 * math.pi)


def _coef_kernel(vec_ref, out_ref):
    vec = vec_ref[...]
    n2 = jnp.sum(vec * vec, axis=1, keepdims=True)
    v = vec * jax.lax.rsqrt(n2)
    x = v[:, 0:1]
    y = v[:, 1:2]
    z = v[:, 2:3]
    sh0 = _S15 * x * z
    sh1 = _S15 * x * y
    sh2c = _S5 * (y * y - 0.5 * (x * x + z * z))
    sh3 = _S15 * y * z
    sh4 = (_S15 / 2.0) * (z * z - x * x)
    ones = jnp.ones_like(x)
    pad = jnp.zeros((vec.shape[0], 10), jnp.float32)
    out_ref[...] = jnp.concatenate(
        [ones, _INV_S4PI * sh0, _INV_S4PI * sh1, _INV_S4PI * sh2c,
         _INV_S4PI * sh3, _INV_S4PI * sh4, pad], axis=1)


def _edge_coefs(egde_vec):
    return pl.pallas_call(
        _coef_kernel,
        grid=(_E // _BE,),
        in_specs=[pl.BlockSpec((_BE, 3), lambda i: (i, 0))],
        out_specs=pl.BlockSpec((_BE, 16), lambda i: (i, 0)),
        out_shape=jax.ShapeDtypeStruct((_E, 16), jnp.float32),
    )(egde_vec)


def _sc_body(x_hbm, coef_hbm, idx_hbm, zeros_hbm, acc_hbm,
             idxvA, xvA, cfvA, idxvB, xvB, cfvB, payv, shacc, semA, semB):
    c = lax.axis_index("c")
    s = lax.axis_index("s")
    ebase = s * _EPT
    rbase = s * _ROWS

    def pinit(e, _):
        payv[e, pl.ds(6 * _CH, 16)] = jnp.ones((16,), jnp.float32)
        return 0
    lax.fori_loop(0, _EB, pinit, 0, unroll=False)

    for p in range(_PASS):
        chunk = c * _PASS + p
        ch0 = chunk * _CH

        def issue(b, idxv, xv, cfv, sem):
            e0 = ebase + b * _EB
            pltpu.async_copy(idx_hbm.at[pl.ds(e0, _EB)], idxv, sem)
            pltpu.async_copy(x_hbm.at[pl.ds(e0, _EB), pl.ds(ch0, _CH)],
                             xv, sem)
            pltpu.async_copy(coef_hbm.at[pl.ds(e0, _EB), :], cfv, sem)

        def wait(b, idxv, xv, cfv, sem):
            e0 = ebase + b * _EB
            pltpu.make_async_copy(idx_hbm.at[pl.ds(e0, _EB)], idxv,
                                  sem).wait()
            pltpu.make_async_copy(x_hbm.at[pl.ds(e0, _EB), pl.ds(ch0, _CH)],
                                  xv, sem).wait()
            pltpu.make_async_copy(coef_hbm.at[pl.ds(e0, _EB), :], cfv,
                                  sem).wait()

        def work(idxv, xv, cfv):
            def edge(e, _):
                row = xv[e, pl.ds(0, 16)]
                cv = cfv[e, pl.ds(0, 16)]
                for k in range(6):
                    payv[e, pl.ds(k * _CH, 16)] = row * cv[k]
                return 0
            lax.fori_loop(0, _EB, edge, 0, unroll=4)
            pltpu.sync_copy(payv, shacc.at[idxv], add=True)

        pltpu.sync_copy(zeros_hbm, shacc.at[pl.ds(rbase, _ROWS), :])
        plsc.subcore_barrier()

        issue(0, idxvA, xvA, cfvA, semA)

        def pair(i, _):
            b0 = 2 * i
            wait(b0, idxvA, xvA, cfvA, semA)
            issue(b0 + 1, idxvB, xvB, cfvB, semB)
            work(idxvA, xvA, cfvA)
            wait(b0 + 1, idxvB, xvB, cfvB, semB)
            issue(b0 + 2, idxvA, xvA, cfvA, semA)
            work(idxvB, xvB, cfvB)
            return 0
        lax.fori_loop(0, (_NBLK - 1) // 2, pair, 0, unroll=False)

        wait(_NBLK - 1, idxvA, xvA, cfvA, semA)
        work(idxvA, xvA, cfvA)

        plsc.subcore_barrier()
        pltpu.sync_copy(shacc.at[pl.ds(rbase, _ROWS), :],
                        acc_hbm.at[chunk, pl.ds(rbase, _ROWS), :])
        plsc.subcore_barrier()


def _sc_segment_sums(x_edge, coef, edge_index):
    zeros = jnp.zeros((_ROWS, _PCOLS), jnp.float32)
    mesh = plsc.VectorSubcoreMesh(core_axis_name="c", subcore_axis_name="s")
    f = pl.kernel(
        _sc_body,
        out_type=jax.ShapeDtypeStruct((_NCHUNK, _NP, _PCOLS), jnp.float32),
        mesh=mesh,
        scratch_types=[
            pltpu.VMEM((_EB,), jnp.int32),
            pltpu.VMEM((_EB, _CH), jnp.float32),
            pltpu.VMEM((_EB, 16), jnp.float32),
            pltpu.VMEM((_EB,), jnp.int32),
            pltpu.VMEM((_EB, _CH), jnp.float32),
            pltpu.VMEM((_EB, 16), jnp.float32),
            pltpu.VMEM((_EB, _PCOLS), jnp.float32),
            pltpu.VMEM_SHARED((_NP, _PCOLS), jnp.float32),
            pltpu.SemaphoreType.DMA,
            pltpu.SemaphoreType.DMA,
        ],
        compiler_params=pltpu.CompilerParams(use_tc_tiling_on_sc=False),
    )
    return f(x_edge, coef, edge_index, zeros)


def _mlp_kernel(acc_ref, bid_ref, ws1_ref, bs1_ref, ws2_ref,
                wi1_ref, bi1_ref, wi2_ref, out_ref, accg_ref):
    step = pl.program_id(0)

    @pl.when(step == 0)
    def _():
        accg_ref[...] = jnp.zeros_like(accg_ref)

    inv = 1.0 / jnp.maximum(acc_ref[0, :, 6 * _CH:6 * _CH + 1], 1.0)

    feats = []
    for k in range(6):
        feats.append(jnp.concatenate(
            [acc_ref[ch, :, k * _CH:(k + 1) * _CH] for ch in range(_NCHUNK)],
            axis=1))

    s_in = feats[0] * inv
    h = s_in @ ws1_ref[...] + bs1_ref[...]
    h = h * jax.nn.sigmoid(h)
    scal = h @ ws2_ref[...]

    i_in = jnp.concatenate(
        [(feats[k + 1] * inv)[:, None, :] for k in range(5)],
        axis=1).reshape(_BN * 5, _D)
    hi = i_in @ wi1_ref[...] + bi1_ref[...]
    hi = hi * jax.nn.sigmoid(hi)
    irr = (hi @ wi2_ref[...]).reshape(_BN, 5)

    vals = jnp.concatenate([scal, irr, jnp.ones_like(scal)], axis=1)
    onehot = (bid_ref[0, :, :] == jax.lax.broadcasted_iota(
        jnp.int32, (_BN, _G), 1)).astype(jnp.float32)
    accg_ref[...] += jax.lax.dot_general(
        onehot, vals, (((0,), (0,)), ((), ())),
        preferred_element_type=jnp.float32)

    @pl.when(step == pl.num_programs(0) - 1)
    def _():
        g = accg_ref[...]
        gcnt = jnp.maximum(g[:, 6:7], 1.0)
        out_ref[...] = g[:, 0:6] / gcnt


def _node_to_graph(acc, batch_ids, Ws1, bs1, Ws2, Wi1, bi1, Wi2):
    bid3 = batch_ids.astype(jnp.int32).reshape(_N // _BN, _BN, 1)
    return pl.pallas_call(
        _mlp_kernel,
        grid=(_N // _BN,),
        in_specs=[
            pl.BlockSpec((_NCHUNK, _BN, _PCOLS), lambda i: (0, i, 0)),
            pl.BlockSpec((1, _BN, 1), lambda i: (i, 0, 0)),
            pl.BlockSpec((_D, _D), lambda i: (0, 0)),
            pl.BlockSpec((1, _D), lambda i: (0, 0)),
            pl.BlockSpec((_D, 1), lambda i: (0, 0)),
            pl.BlockSpec((_D, _D), lambda i: (0, 0)),
            pl.BlockSpec((1, _D), lambda i: (0, 0)),
            pl.BlockSpec((_D, 1), lambda i: (0, 0)),
        ],
        out_specs=pl.BlockSpec((_G, 6), lambda i: (0, 0)),
        out_shape=jax.ShapeDtypeStruct((_G, 6), jnp.float32),
        scratch_shapes=[pltpu.VMEM((_G, 7), jnp.float32)],
    )(acc, bid3, Ws1, bs1, Ws2, Wi1, bi1, Wi2)


@jax.jit
def kernel(x_edge, egde_vec, edge_index, batch_ids, Ws1, bs1, Ws2, bs2,
           Wi1, bi1, Wi2, bi2):
    coef = _edge_coefs(egde_vec)
    acc = _sc_segment_sums(x_edge, coef, edge_index.astype(jnp.int32))
    out = _node_to_graph(acc, batch_ids,
                         Ws1.T, bs1.reshape(1, _D), Ws2.T,
                         Wi1.T, bi1.reshape(1, _D), Wi2.T)
    scalar = out[:, 0] + bs2[0] + 0.0
    irrep2 = out[:, 1:6] + bi2[0]
    return (scalar, irrep2)


# speedup vs baseline: 1.0196x; 1.0196x over previous
import functools
import math

import jax
import jax.numpy as jnp
from jax import lax
from jax.experimental import pallas as pl
from jax.experimental.pallas import tpu as pltpu
from jax.experimental.pallas import tpu_sc as plsc

_N = 10000
_G = 64
_E = 160000
_D = 128

_NC = 2
_NS = 16
_CH = 16
_NCHUNK = 8
_PASS = _NCHUNK // _NC
_EB = 80
_EPT = _E // _NS
_NBLK = _EPT // _EB
_NP = 10240
_ROWS = _NP // _NS
_PCOLS = 6 * _CH + 16

_BE = 10000
_BN = 1000
_S15 = math.sqrt(15.0)
_S5 = math.sqrt(5.0)
_INV_S4PI = 1.0 / math.sqrt(4.0 * math.pi)


def _coef_kernel(vec_ref, out_ref):
    vec = vec_ref[...]
    n2 = jnp.sum(vec * vec, axis=1, keepdims=True)
    v = vec * jax.lax.rsqrt(n2)
    x = v[:, 0:1]
    y = v[:, 1:2]
    z = v[:, 2:3]
    sh0 = _S15 * x * z
    sh1 = _S15 * x * y
    sh2c = _S5 * (y * y - 0.5 * (x * x + z * z))
    sh3 = _S15 * y * z
    sh4 = (_S15 / 2.0) * (z * z - x * x)
    ones = jnp.ones_like(x)
    pad = jnp.zeros((vec.shape[0], 10), jnp.float32)
    out_ref[...] = jnp.concatenate(
        [ones, _INV_S4PI * sh0, _INV_S4PI * sh1, _INV_S4PI * sh2c,
         _INV_S4PI * sh3, _INV_S4PI * sh4, pad], axis=1)


def _edge_coefs(egde_vec):
    return pl.pallas_call(
        _coef_kernel,
        grid=(_E // _BE,),
        in_specs=[pl.BlockSpec((_BE, 3), lambda i: (i, 0))],
        out_specs=pl.BlockSpec((_BE, 16), lambda i: (i, 0)),
        out_shape=jax.ShapeDtypeStruct((_E, 16), jnp.float32),
    )(egde_vec)


def _sc_body(x_hbm, coef_hbm, idx_hbm, zeros_hbm, acc_hbm,
             idxvA, xvA, cfvA, idxvB, xvB, cfvB, payv, shacc, semA, semB):
    c = lax.axis_index("c")
    s = lax.axis_index("s")
    ebase = s * _EPT
    rbase = s * _ROWS

    def pinit(e, _):
        payv[e, pl.ds(6 * _CH, 16)] = jnp.ones((16,), jnp.float32)
        return 0
    lax.fori_loop(0, _EB, pinit, 0, unroll=False)

    for p in range(_PASS):
        chunk = c * _PASS + p
        ch0 = chunk * _CH

        def issue(b, idxv, xv, cfv, sem):
            e0 = ebase + b * _EB
            pltpu.async_copy(idx_hbm.at[pl.ds(e0, _EB)], idxv, sem)
            pltpu.async_copy(x_hbm.at[pl.ds(e0, _EB), pl.ds(ch0, _CH)],
                             xv, sem)
            pltpu.async_copy(coef_hbm.at[pl.ds(e0, _EB), :], cfv, sem)

        def wait(b, idxv, xv, cfv, sem):
            e0 = ebase + b * _EB
            pltpu.make_async_copy(idx_hbm.at[pl.ds(e0, _EB)], idxv,
                                  sem).wait()
            pltpu.make_async_copy(x_hbm.at[pl.ds(e0, _EB), pl.ds(ch0, _CH)],
                                  xv, sem).wait()
            pltpu.make_async_copy(coef_hbm.at[pl.ds(e0, _EB), :], cfv,
                                  sem).wait()

        def work(idxv, xv, cfv):
            def edge(e, _):
                row = xv[e, pl.ds(0, 16)]
                cv = cfv[e, pl.ds(0, 16)]
                for k in range(6):
                    payv[e, pl.ds(k * _CH, 16)] = row * cv[k]
                return 0
            lax.fori_loop(0, _EB, edge, 0, unroll=False)
            pltpu.sync_copy(payv, shacc.at[idxv], add=True)

        pltpu.sync_copy(zeros_hbm, shacc.at[pl.ds(rbase, _ROWS), :])
        plsc.subcore_barrier()

        issue(0, idxvA, xvA, cfvA, semA)

        def pair(i, _):
            b0 = 2 * i
            wait(b0, idxvA, xvA, cfvA, semA)
            issue(b0 + 1, idxvB, xvB, cfvB, semB)
            work(idxvA, xvA, cfvA)
            wait(b0 + 1, idxvB, xvB, cfvB, semB)
            issue(b0 + 2, idxvA, xvA, cfvA, semA)
            work(idxvB, xvB, cfvB)
            return 0
        lax.fori_loop(0, (_NBLK - 1) // 2, pair, 0, unroll=False)

        wait(_NBLK - 1, idxvA, xvA, cfvA, semA)
        work(idxvA, xvA, cfvA)

        plsc.subcore_barrier()
        pltpu.sync_copy(shacc.at[pl.ds(rbase, _ROWS), :],
                        acc_hbm.at[chunk, pl.ds(rbase, _ROWS), :])
        plsc.subcore_barrier()


def _sc_segment_sums(x_edge, coef, edge_index):
    zeros = jnp.zeros((_ROWS, _PCOLS), jnp.float32)
    mesh = plsc.VectorSubcoreMesh(core_axis_name="c", subcore_axis_name="s")
    f = pl.kernel(
        _sc_body,
        out_type=jax.ShapeDtypeStruct((_NCHUNK, _NP, _PCOLS), jnp.float32),
        mesh=mesh,
        scratch_types=[
            pltpu.VMEM((_EB,), jnp.int32),
            pltpu.VMEM((_EB, _CH), jnp.float32),
            pltpu.VMEM((_EB, 16), jnp.float32),
            pltpu.VMEM((_EB,), jnp.int32),
            pltpu.VMEM((_EB, _CH), jnp.float32),
            pltpu.VMEM((_EB, 16), jnp.float32),
            pltpu.VMEM((_EB, _PCOLS), jnp.float32),
            pltpu.VMEM_SHARED((_NP, _PCOLS), jnp.float32),
            pltpu.SemaphoreType.DMA,
            pltpu.SemaphoreType.DMA,
        ],
        compiler_params=pltpu.CompilerParams(use_tc_tiling_on_sc=False),
    )
    return f(x_edge, coef, edge_index, zeros)


def _mlp_kernel(acc_ref, bid_ref, ws1_ref, bs1_ref, ws2_ref,
                wi1_ref, bi1_ref, wi2_ref, out_ref, accg_ref):
    step = pl.program_id(0)

    @pl.when(step == 0)
    def _():
        accg_ref[...] = jnp.zeros_like(accg_ref)

    inv = 1.0 / jnp.maximum(acc_ref[0, :, 6 * _CH:6 * _CH + 1], 1.0)

    feats = []
    for k in range(6):
        feats.append(jnp.concatenate(
            [acc_ref[ch, :, k * _CH:(k + 1) * _CH] for ch in range(_NCHUNK)],
            axis=1))

    s_in = feats[0] * inv
    h = s_in @ ws1_ref[...] + bs1_ref[...]
    h = h * jax.nn.sigmoid(h)
    scal = h @ ws2_ref[...]

    i_in = jnp.concatenate(
        [(feats[k + 1] * inv)[:, None, :] for k in range(5)],
        axis=1).reshape(_BN * 5, _D)
    hi = i_in @ wi1_ref[...] + bi1_ref[...]
    hi = hi * jax.nn.sigmoid(hi)
    irr = (hi @ wi2_ref[...]).reshape(_BN, 5)

    vals = jnp.concatenate([scal, irr, jnp.ones_like(scal)], axis=1)
    onehot = (bid_ref[0, :, :] == jax.lax.broadcasted_iota(
        jnp.int32, (_BN, _G), 1)).astype(jnp.float32)
    accg_ref[...] += jax.lax.dot_general(
        onehot, vals, (((0,), (0,)), ((), ())),
        preferred_element_type=jnp.float32)

    @pl.when(step == pl.num_programs(0) - 1)
    def _():
        g = accg_ref[...]
        gcnt = jnp.maximum(g[:, 6:7], 1.0)
        out_ref[...] = g[:, 0:6] / gcnt


def _node_to_graph(acc, batch_ids, Ws1, bs1, Ws2, Wi1, bi1, Wi2):
    bid3 = batch_ids.astype(jnp.int32).reshape(_N // _BN, _BN, 1)
    return pl.pallas_call(
        _mlp_kernel,
        grid=(_N // _BN,),
        in_specs=[
            pl.BlockSpec((_NCHUNK, _BN, _PCOLS), lambda i: (0, i, 0)),
            pl.BlockSpec((1, _BN, 1), lambda i: (i, 0, 0)),
            pl.BlockSpec((_D, _D), lambda i: (0, 0)),
            pl.BlockSpec((1, _D), lambda i: (0, 0)),
            pl.BlockSpec((_D, 1), lambda i: (0, 0)),
            pl.BlockSpec((_D, _D), lambda i: (0, 0)),
            pl.BlockSpec((1, _D), lambda i: (0, 0)),
            pl.BlockSpec((_D, 1), lambda i: (0, 0)),
        ],
        out_specs=pl.BlockSpec((_G, 6), lambda i: (0, 0)),
        out_shape=jax.ShapeDtypeStruct((_G, 6), jnp.float32),
        scratch_shapes=[pltpu.VMEM((_G, 7), jnp.float32)],
    )(acc, bid3, Ws1, bs1, Ws2, Wi1, bi1, Wi2)


@jax.jit
def kernel(x_edge, egde_vec, edge_index, batch_ids, Ws1, bs1, Ws2, bs2,
           Wi1, bi1, Wi2, bi2):
    coef = _edge_coefs(egde_vec)
    acc = _sc_segment_sums(x_edge, coef, edge_index.astype(jnp.int32))
    out = _node_to_graph(acc, batch_ids,
                         Ws1.T, bs1.reshape(1, _D), Ws2.T,
                         Wi1.T, bi1.reshape(1, _D), Wi2.T)
    scalar = out[:, 0] + bs2[0] + 0.0
    irrep2 = out[:, 1:6] + bi2[0]
    return (scalar, irrep2)
